# baseline (speedup 1.0000x reference)
import functools
import math

import jax
import jax.numpy as jnp
from jax import lax
from jax.experimental import pallas as pl
from jax.experimental.pallas import tpu as pltpu
from jax.experimental.pallas import tpu_sc as plsc

_D = 512
_G = 1000
_P = 8
_M = 16
_TOPK = 5
_ATHR = math.sin(1000.0 / (2.0 * 6371.0)) ** 2

_NC = 2
_NS = 16
_NW = _NC * _NS


def _gather_rows(table, idx, *, chunk=80, unroll=10):
    n, = idx.shape
    _, d = table.shape
    rpw = n // _NW
    assert rpw * _NW == n and rpw % chunk == 0 and chunk % 8 == 0
    nchunks = rpw // chunk
    mesh = plsc.VectorSubcoreMesh(core_axis_name="c", subcore_axis_name="s")

    @functools.partial(
        pl.kernel,
        mesh=mesh,
        compiler_params=pltpu.CompilerParams(use_tc_tiling_on_sc=(d % 128 == 0)),
        out_type=jax.ShapeDtypeStruct((n, d), table.dtype),
        scratch_types=[
            pltpu.VMEM((rpw,), jnp.int32),
            pltpu.VMEM((chunk, d), table.dtype),
            pltpu.VMEM((chunk, d), table.dtype),
            pltpu.SemaphoreType.DMA,
            pltpu.SemaphoreType.DMA,
        ],
    )
    def k(table_hbm, idx_hbm, out_hbm, idx_v, buf0, buf1, sem0, sem1):
        wid = lax.axis_index("s") * _NC + lax.axis_index("c")
        base = wid * rpw
        pltpu.sync_copy(idx_hbm.at[pl.ds(base, rpw)], idx_v)
        bufs = (buf0, buf1)
        sems = (sem0, sem1)

        def run_group(g, nin):
            cps = {}

            def start(j):
                ci = g * unroll + j
                cps[j] = pltpu.async_copy(
                    table_hbm.at[idx_v.at[pl.ds(ci * chunk, chunk)]],
                    bufs[j % 2],
                    sems[j % 2],
                )

            start(0)
            for j in range(nin):
                if j + 1 < nin:
                    start(j + 1)
                cps[j].wait()
                ci = g * unroll + j
                pltpu.sync_copy(
                    bufs[j % 2],
                    out_hbm.at[pl.ds(base + ci * chunk, chunk)],
                )

        full, rem = divmod(nchunks, unroll)
        if full > 1:
            def body(g, c):
                run_group(g, unroll)
                return c
            lax.fori_loop(0, full, body, 0)
        elif full == 1:
            run_group(0, unroll)
        if rem:
            run_group(full, rem)

    return k(table, idx)


def _mean_members(rows):
    gp = rows.shape[0]
    blk = 200

    def kfn(r_ref, o_ref):
        o_ref[...] = jnp.mean(r_ref[...], axis=1)

    return pl.pallas_call(
        kfn,
        grid=(gp // blk,),
        in_specs=[pl.BlockSpec((blk, _M, _D), lambda i: (i, 0, 0))],
        out_specs=pl.BlockSpec((blk, _D), lambda i: (i, 0)),
        out_shape=jax.ShapeDtypeStruct((gp, _D), jnp.float32),
    )(rows)


def _proto_argmin(rows2, emb):
    b = emb.shape[0]
    bb = 128

    def kfn(r_ref, e_ref, bp_ref, bd_ref):
        r = r_ref[...]
        e = e_ref[...]
        dd = r - e[:, None, None, :]
        s = jnp.sqrt(jnp.sum(dd * dd, axis=-1) + 1e-12)
        best = s[:, :, 0]
        bi = jnp.zeros(best.shape, jnp.int32)
        for p in range(1, _P):
            c = s[:, :, p]
            lt = c < best
            bi = jnp.where(lt, p, bi)
            best = jnp.where(lt, c, best)
        bp_ref[...] = bi
        bd_ref[...] = best

    return pl.pallas_call(
        kfn,
        grid=(b // bb,),
        in_specs=[
            pl.BlockSpec((bb, _TOPK, _P, _D), lambda i: (i, 0, 0, 0)),
            pl.BlockSpec((bb, _D), lambda i: (i, 0)),
        ],
        out_specs=[
            pl.BlockSpec((bb, _TOPK), lambda i: (i, 0)),
            pl.BlockSpec((bb, _TOPK), lambda i: (i, 0)),
        ],
        out_shape=[
            jax.ShapeDtypeStruct((b, _TOPK), jnp.int32),
            jax.ShapeDtypeStruct((b, _TOPK), jnp.float32),
        ],
    )(rows2, emb)


def _member_argmin(rows3, emb, bm):
    b = emb.shape[0]
    bb = 64

    def kfn(r_ref, e_ref, bm_ref, bg_ref):
        r = r_ref[...]
        e = e_ref[...]
        dd = r - e[:, None, None, :]
        s = jnp.sqrt(jnp.sum(dd * dd, axis=-1) + 1e-12)
        best = s[:, :, 0]
        bi = jnp.zeros(best.shape, jnp.int32)
        for m in range(1, _M):
            c = s[:, :, m]
            lt = c < best
            bi = jnp.where(lt, m, bi)
            best = jnp.where(lt, c, best)
        bmv = bm_ref[...]
        bg = bmv[:, :, 0]
        for m in range(1, _M):
            bg = jnp.where(bi == m, bmv[:, :, m], bg)
        bg_ref[...] = bg

    return pl.pallas_call(
        kfn,
        grid=(b // bb,),
        in_specs=[
            pl.BlockSpec((bb, _TOPK, _M, _D), lambda i: (i, 0, 0, 0)),
            pl.BlockSpec((bb, _D), lambda i: (i, 0)),
            pl.BlockSpec((bb, _TOPK, _M), lambda i: (i, 0, 0)),
        ],
        out_specs=pl.BlockSpec((bb, _TOPK), lambda i: (i, 0)),
        out_shape=jax.ShapeDtypeStruct((b, _TOPK), jnp.int32),
    )(rows3, emb, bm)


def _finish(bd, cp5, cand5, latk, lngk, ipreds, temp):
    b = bd.shape[0]

    def kfn(bd_ref, cp_ref, cd_ref, la_ref, lo_ref, ip_ref, t_ref,
            olat_ref, olng_ref, ogc_ref):
        scores = -bd_ref[...]
        t = t_ref[0, 0]
        ex = jnp.exp(scores / t)
        probs = ex / jnp.sum(ex, axis=-1, keepdims=True)
        fp = cp_ref[...] * probs
        best = fp[:, 0]
        bi = jnp.zeros(best.shape, jnp.int32)
        for k2 in range(1, _TOPK):
            c = fp[:, k2]
            gt = c > best
            bi = jnp.where(gt, k2, bi)
            best = jnp.where(gt, c, best)
        la = la_ref[...]
        lo = lo_ref[...]
        rlat = la[:, 0]
        rlng = lo[:, 0]
        for k2 in range(1, _TOPK):
            rlat = jnp.where(bi == k2, la[:, k2], rlat)
            rlng = jnp.where(bi == k2, lo[:, k2], rlng)
        deg = jnp.float32(math.pi / 180.0)
        lat1 = ip_ref[:, 0] * deg
        lng1 = ip_ref[:, 1] * deg
        lat2 = rlat * deg
        lng2 = rlng * deg
        sdlat = jnp.sin((lat2 - lat1) * 0.5)
        sdlng = jnp.sin((lng2 - lng1) * 0.5)
        a = sdlat * sdlat + jnp.cos(lat1) * jnp.cos(lat2) * sdlng * sdlng
        fi = jnp.where(a > jnp.float32(_ATHR), 0, bi)
        flat = la[:, 0]
        flng = lo[:, 0]
        cd = cd_ref[...]
        gc = cd[:, 0]
        for k2 in range(1, _TOPK):
            sel = fi == k2
            flat = jnp.where(sel, la[:, k2], flat)
            flng = jnp.where(sel, lo[:, k2], flng)
            gc = jnp.where(sel, cd[:, k2], gc)
        olat_ref[...] = flat
        olng_ref[...] = flng
        ogc_ref[...] = gc

    return pl.pallas_call(
        kfn,
        out_shape=[
            jax.ShapeDtypeStruct((b,), jnp.float32),
            jax.ShapeDtypeStruct((b,), jnp.float32),
            jax.ShapeDtypeStruct((b,), cand5.dtype),
        ],
    )(bd, cp5, cand5, latk, lngk, ipreds, temp)


def kernel(embedding, initial_preds, candidate_cells, candidate_probs,
           embeddings, proto_indices, dataset_latlng, temperature):
    b, d = embedding.shape
    idx1 = proto_indices.reshape(-1).astype(jnp.int32)
    rows1 = _gather_rows(embeddings, idx1)
    pm = _mean_members(rows1.reshape(_G * _P, _M, d))
    cand = candidate_cells[:, :_TOPK].astype(jnp.int32)
    idx2 = (cand[:, :, None] * _P
            + jnp.arange(_P, dtype=jnp.int32)[None, None, :]).reshape(-1)
    rows2 = _gather_rows(pm, idx2).reshape(b, _TOPK, _P, d)
    bp, bd = _proto_argmin(rows2, embedding)
    idx3 = (cand * _P + bp).reshape(-1)
    bm = _gather_rows(proto_indices.reshape(_G * _P, _M).astype(jnp.int32), idx3)
    rows3 = _gather_rows(embeddings, bm.reshape(-1)).reshape(b, _TOPK, _M, d)
    bg = _member_argmin(rows3, embedding, bm.reshape(b, _TOPK, _M))
    llpad = jnp.pad(dataset_latlng, ((0, 0), (0, 14)))
    crows = _gather_rows(llpad, bg.reshape(-1)).reshape(b, _TOPK, 16)
    latk = crows[:, :, 0]
    lngk = crows[:, :, 1]
    return _finish(bd, candidate_probs[:, :_TOPK], cand, latk, lngk,
                   initial_preds, jnp.reshape(temperature, (1, 1)))
